# baseline (speedup 1.0000x reference)
import functools

import jax
import jax.numpy as jnp
from jax import lax
from jax.experimental import pallas as pl
from jax.experimental.pallas import tpu as pltpu
from jax.experimental.pallas import tpu_sc as plsc

ROWS = 64
COLS = 32768
LANES = 16
NUM_CORES = 2
NUM_SUBCORES = 16
NUM_WORKERS = NUM_CORES * NUM_SUBCORES
ROWS_PER_WORKER = ROWS // NUM_WORKERS
NVREGS = COLS // LANES
GROUP_VREGS = 4
GROUP_COLS = GROUP_VREGS * LANES
NGROUPS = COLS // GROUP_COLS
BISECT_ITERS = 15
RESIDENT_VREGS = 16
RESIDENT_COLS = RESIDENT_VREGS * LANES

_MESH = plsc.VectorSubcoreMesh(core_axis_name="c", subcore_axis_name="s")

_GATHER_DNUMS = lax.GatherDimensionNumbers(
    offset_dims=(), collapsed_slice_dims=(0,), start_index_map=(0,)
)


def _shuffle(v, sh):
    idx = jnp.bitwise_xor(lax.iota(jnp.int32, LANES), sh)
    return lax.gather(
        v,
        idx[:, None],
        dimension_numbers=_GATHER_DNUMS,
        slice_sizes=(1,),
        mode=lax.GatherScatterMode.PROMISE_IN_BOUNDS,
    )


def _allmax(v):
    for sh in (8, 4, 2, 1):
        v = jnp.maximum(v, _shuffle(v, sh))
    return v


def _allsum(v):
    for sh in (8, 4, 2, 1):
        v = v + _shuffle(v, sh)
    return v


def _allmax_i(v):
    for sh in (8, 4, 2, 1):
        v = jnp.maximum(v, _shuffle(v, sh))
    return v


def _allmin_i(v):
    for sh in (8, 4, 2, 1):
        v = jnp.minimum(v, _shuffle(v, sh))
    return v


def _max_pass(row_v, lo_vreg, hi_vreg, acc_init):
    @plsc.parallel_loop(lo_vreg, hi_vreg, step=8, unroll=1, carry=acc_init)
    def acc(j, acc_c):
        vs = [row_v[pl.ds((j + k) * LANES, LANES)] for k in range(8)]
        m01 = jnp.maximum(vs[0], vs[1])
        m23 = jnp.maximum(vs[2], vs[3])
        m45 = jnp.maximum(vs[4], vs[5])
        m67 = jnp.maximum(vs[6], vs[7])
        m = jnp.maximum(jnp.maximum(m01, m23), jnp.maximum(m45, m67))
        return jnp.maximum(acc_c, m)

    return acc


def _row_tau(row_v, cand_v, acc):
    f32 = jnp.float32
    maxv = _allmax(acc)
    thr = maxv - 1.0

    lane_id = lax.iota(jnp.int32, LANES)
    sixteen = jnp.full((LANES,), LANES, jnp.int32)
    zero_i = jnp.zeros((LANES,), jnp.int32)

    with jax.named_scope("compact"):
        @plsc.parallel_loop(0, NVREGS, step=1, unroll=2,
                            carry=jnp.zeros((LANES,), jnp.int32))
        def cnt16(j, cnt_c):
            v = row_v[pl.ds(j * LANES, LANES)]
            m = v > thr
            plsc.store_scatter(cand_v, [cnt_c + lane_id], v, mask=m)
            return cnt_c + jnp.where(m, sixteen, zero_i)

    targ16 = jnp.maximum(_allmax_i(cnt16), jnp.full((LANES,), RESIDENT_COLS,
                                                    jnp.int32))
    minc16 = _allmin_i(cnt16)
    n_pad = ((targ16 - minc16) // LANES)[0]

    def pad_body(_, cnt_c):
        m = cnt_c < targ16
        plsc.store_scatter(cand_v, [cnt_c + lane_id], thr, mask=m)
        return cnt_c + jnp.where(m, sixteen, zero_i)

    lax.fori_loop(0, n_pad, pad_body, cnt16)
    nv = (targ16 // LANES)[0]

    cvs = [cand_v[pl.ds(i * LANES, LANES)] for i in range(RESIDENT_VREGS)]

    def bis_body(_, carry):
        lo, hi = carry
        mid = 0.5 * (lo + hi)
        sacc = jnp.zeros((LANES,), f32)
        for cv in cvs:
            sacc = sacc + jnp.maximum(cv - mid, 0.0)

        def s_body(j, sa):
            v = cand_v[pl.ds(j * LANES, LANES)]
            return sa + jnp.maximum(v - mid, 0.0)

        sacc = lax.fori_loop(RESIDENT_VREGS, nv, s_body, sacc)
        pred = _allsum(sacc) >= 1.0
        return jnp.where(pred, mid, lo), jnp.where(pred, hi, mid)

    with jax.named_scope("bisect"):
        lo, _ = lax.fori_loop(0, BISECT_ITERS, bis_body, (thr, maxv))

    sacc = jnp.zeros((LANES,), f32)
    cacc = jnp.zeros((LANES,), f32)
    for cv in cvs:
        m = cv > lo
        sacc = sacc + jnp.where(m, cv, 0.0)
        cacc = cacc + m.astype(f32)

    def rf_body(j, carry):
        sa, ca = carry
        v = cand_v[pl.ds(j * LANES, LANES)]
        m = v > lo
        return sa + jnp.where(m, v, 0.0), ca + m.astype(f32)

    sacc, cacc = lax.fori_loop(RESIDENT_VREGS, nv, rf_body, (sacc, cacc))
    t_vec = (_allsum(sacc) - 1.0) / _allsum(cacc)
    return t_vec


def _out_pass(row_v, t_vec, lo_vreg, hi_vreg):
    @plsc.parallel_loop(lo_vreg, hi_vreg, step=8, unroll=1)
    def _(j):
        for k in range(8):
            v = row_v[pl.ds((j + k) * LANES, LANES)]
            row_v[pl.ds((j + k) * LANES, LANES)] = jnp.maximum(v - t_vec, 0.0)


NCHUNKS = 4
CHUNK_COLS = COLS // NCHUNKS
CHUNK_VREGS = NVREGS // NCHUNKS


@functools.partial(
    pl.kernel,
    out_type=jax.ShapeDtypeStruct((ROWS, COLS), jnp.float32),
    mesh=_MESH,
    scratch_types=[
        pltpu.VMEM((COLS,), jnp.float32),
        pltpu.VMEM((COLS,), jnp.float32),
        pltpu.VMEM((COLS + LANES,), jnp.float32),
        [pltpu.SemaphoreType.DMA] * NCHUNKS,
        pltpu.SemaphoreType.DMA,
        pltpu.SemaphoreType.DMA,
    ],
    compiler_params=pltpu.CompilerParams(needs_layout_passes=False),
)
def _sparsemax_sc(x_hbm, o_hbm, row0_v, row1_v, cand_v,
                  sems_c, sem_i1, sem_o0):
    wid = lax.axis_index("s") * NUM_CORES + lax.axis_index("c")
    r0 = wid * ROWS_PER_WORKER
    r1 = r0 + 1
    cps0 = [
        pltpu.async_copy(
            x_hbm.at[r0, pl.ds(k * CHUNK_COLS, CHUNK_COLS)],
            row0_v.at[pl.ds(k * CHUNK_COLS, CHUNK_COLS)],
            sems_c[k],
        )
        for k in range(NCHUNKS)
    ]
    acc = jnp.full((LANES,), -3.4e38, jnp.float32)
    with jax.named_scope("max0"):
        for k in range(NCHUNKS):
            cps0[k].wait()
            acc = _max_pass(row0_v, k * CHUNK_VREGS, (k + 1) * CHUNK_VREGS, acc)
    cp_i1 = pltpu.async_copy(x_hbm.at[r1], row1_v, sem_i1)
    with jax.named_scope("tau0"):
        t0 = _row_tau(row0_v, cand_v, acc)
    with jax.named_scope("out0"):
        _out_pass(row0_v, t0, 0, NVREGS)
    cp_o0 = pltpu.async_copy(row0_v, o_hbm.at[r0], sem_o0)
    with jax.named_scope("wait_i1"):
        cp_i1.wait()
    with jax.named_scope("max1"):
        acc1 = _max_pass(row1_v, 0, NVREGS,
                         jnp.full((LANES,), -3.4e38, jnp.float32))
    with jax.named_scope("tau1"):
        t1 = _row_tau(row1_v, cand_v, acc1)
    cps1 = []
    for k in range(NCHUNKS):
        _out_pass(row1_v, t1, k * CHUNK_VREGS, (k + 1) * CHUNK_VREGS)
        cps1.append(
            pltpu.async_copy(
                row1_v.at[pl.ds(k * CHUNK_COLS, CHUNK_COLS)],
                o_hbm.at[r1, pl.ds(k * CHUNK_COLS, CHUNK_COLS)],
                sems_c[k],
            )
        )
    cp_o0.wait()
    for cp in cps1:
        cp.wait()


def kernel(inputs):
    return _sparsemax_sc(inputs)


# speedup vs baseline: 1.1271x; 1.1271x over previous
import functools

import jax
import jax.numpy as jnp
from jax import lax
from jax.experimental import pallas as pl
from jax.experimental.pallas import tpu as pltpu
from jax.experimental.pallas import tpu_sc as plsc

ROWS = 64
COLS = 32768
LANES = 16
NUM_CORES = 2
NUM_SUBCORES = 16
NUM_WORKERS = NUM_CORES * NUM_SUBCORES
ROWS_PER_WORKER = ROWS // NUM_WORKERS
NVREGS = COLS // LANES
GROUP_VREGS = 4
GROUP_COLS = GROUP_VREGS * LANES
NGROUPS = COLS // GROUP_COLS
BISECT_ITERS = 15
RESIDENT_VREGS = 16
RESIDENT_COLS = RESIDENT_VREGS * LANES

_MESH = plsc.VectorSubcoreMesh(core_axis_name="c", subcore_axis_name="s")

_GATHER_DNUMS = lax.GatherDimensionNumbers(
    offset_dims=(), collapsed_slice_dims=(0,), start_index_map=(0,)
)


def _shuffle(v, sh):
    idx = jnp.bitwise_xor(lax.iota(jnp.int32, LANES), sh)
    return lax.gather(
        v,
        idx[:, None],
        dimension_numbers=_GATHER_DNUMS,
        slice_sizes=(1,),
        mode=lax.GatherScatterMode.PROMISE_IN_BOUNDS,
    )


def _allmax(v):
    for sh in (8, 4, 2, 1):
        v = jnp.maximum(v, _shuffle(v, sh))
    return v


def _allsum(v):
    for sh in (8, 4, 2, 1):
        v = v + _shuffle(v, sh)
    return v


def _allmax_i(v):
    for sh in (8, 4, 2, 1):
        v = jnp.maximum(v, _shuffle(v, sh))
    return v


def _allmin_i(v):
    for sh in (8, 4, 2, 1):
        v = jnp.minimum(v, _shuffle(v, sh))
    return v


def _max_pass(row_v, lo_vreg, hi_vreg, acc_init):
    @plsc.parallel_loop(lo_vreg, hi_vreg, step=8, unroll=2, carry=acc_init)
    def acc(j, acc_c):
        vs = [row_v[pl.ds((j + k) * LANES, LANES)] for k in range(8)]
        m01 = jnp.maximum(vs[0], vs[1])
        m23 = jnp.maximum(vs[2], vs[3])
        m45 = jnp.maximum(vs[4], vs[5])
        m67 = jnp.maximum(vs[6], vs[7])
        m = jnp.maximum(jnp.maximum(m01, m23), jnp.maximum(m45, m67))
        return jnp.maximum(acc_c, m)

    return acc


def _row_tau(row_v, cand_v, acc):
    f32 = jnp.float32
    maxv = _allmax(acc)
    thr = maxv - 1.0

    lane_id = lax.iota(jnp.int32, LANES)
    sixteen = jnp.full((LANES,), LANES, jnp.int32)
    zero_i = jnp.zeros((LANES,), jnp.int32)

    with jax.named_scope("compact"):
        @plsc.parallel_loop(0, NVREGS, step=1, unroll=16,
                            carry=jnp.zeros((LANES,), jnp.int32))
        def cnt16(j, cnt_c):
            v = row_v[pl.ds(j * LANES, LANES)]
            m = v > thr
            plsc.store_scatter(cand_v, [cnt_c + lane_id], v, mask=m)
            return cnt_c + jnp.where(m, sixteen, zero_i)

    targ16 = jnp.maximum(_allmax_i(cnt16), jnp.full((LANES,), RESIDENT_COLS,
                                                    jnp.int32))
    minc16 = _allmin_i(cnt16)
    n_pad = ((targ16 - minc16) // LANES)[0]

    def pad_body(_, cnt_c):
        m = cnt_c < targ16
        plsc.store_scatter(cand_v, [cnt_c + lane_id], thr, mask=m)
        return cnt_c + jnp.where(m, sixteen, zero_i)

    lax.fori_loop(0, n_pad, pad_body, cnt16)
    nv = (targ16 // LANES)[0]

    cvs = [cand_v[pl.ds(i * LANES, LANES)] for i in range(RESIDENT_VREGS)]

    def bis_body(_, carry):
        lo, hi = carry
        mid = 0.5 * (lo + hi)
        sacc = jnp.zeros((LANES,), f32)
        for cv in cvs:
            sacc = sacc + jnp.maximum(cv - mid, 0.0)

        def s_body(j, sa):
            v = cand_v[pl.ds(j * LANES, LANES)]
            return sa + jnp.maximum(v - mid, 0.0)

        sacc = lax.fori_loop(RESIDENT_VREGS, nv, s_body, sacc)
        pred = _allsum(sacc) >= 1.0
        return jnp.where(pred, mid, lo), jnp.where(pred, hi, mid)

    with jax.named_scope("bisect"):
        lo, _ = lax.fori_loop(0, BISECT_ITERS, bis_body, (thr, maxv))

    sacc = jnp.zeros((LANES,), f32)
    cacc = jnp.zeros((LANES,), f32)
    for cv in cvs:
        m = cv > lo
        sacc = sacc + jnp.where(m, cv, 0.0)
        cacc = cacc + m.astype(f32)

    def rf_body(j, carry):
        sa, ca = carry
        v = cand_v[pl.ds(j * LANES, LANES)]
        m = v > lo
        return sa + jnp.where(m, v, 0.0), ca + m.astype(f32)

    sacc, cacc = lax.fori_loop(RESIDENT_VREGS, nv, rf_body, (sacc, cacc))
    t_vec = (_allsum(sacc) - 1.0) / _allsum(cacc)
    return t_vec


def _out_pass(row_v, t_vec, lo_vreg, hi_vreg):
    @plsc.parallel_loop(lo_vreg, hi_vreg, step=8, unroll=2)
    def _(j):
        for k in range(8):
            v = row_v[pl.ds((j + k) * LANES, LANES)]
            row_v[pl.ds((j + k) * LANES, LANES)] = jnp.maximum(v - t_vec, 0.0)


NCHUNKS = 4
CHUNK_COLS = COLS // NCHUNKS
CHUNK_VREGS = NVREGS // NCHUNKS


@functools.partial(
    pl.kernel,
    out_type=jax.ShapeDtypeStruct((ROWS, COLS), jnp.float32),
    mesh=_MESH,
    scratch_types=[
        pltpu.VMEM((COLS,), jnp.float32),
        pltpu.VMEM((COLS,), jnp.float32),
        pltpu.VMEM((COLS + LANES,), jnp.float32),
        [pltpu.SemaphoreType.DMA] * NCHUNKS,
        pltpu.SemaphoreType.DMA,
        pltpu.SemaphoreType.DMA,
    ],
    compiler_params=pltpu.CompilerParams(needs_layout_passes=False),
)
def _sparsemax_sc(x_hbm, o_hbm, row0_v, row1_v, cand_v,
                  sems_c, sem_i1, sem_o0):
    wid = lax.axis_index("s") * NUM_CORES + lax.axis_index("c")
    r0 = wid * ROWS_PER_WORKER
    r1 = r0 + 1
    cps0 = [
        pltpu.async_copy(
            x_hbm.at[r0, pl.ds(k * CHUNK_COLS, CHUNK_COLS)],
            row0_v.at[pl.ds(k * CHUNK_COLS, CHUNK_COLS)],
            sems_c[k],
        )
        for k in range(NCHUNKS)
    ]
    acc = jnp.full((LANES,), -3.4e38, jnp.float32)
    with jax.named_scope("max0"):
        for k in range(NCHUNKS):
            cps0[k].wait()
            acc = _max_pass(row0_v, k * CHUNK_VREGS, (k + 1) * CHUNK_VREGS, acc)
    cp_i1 = pltpu.async_copy(x_hbm.at[r1], row1_v, sem_i1)
    with jax.named_scope("tau0"):
        t0 = _row_tau(row0_v, cand_v, acc)
    with jax.named_scope("out0"):
        _out_pass(row0_v, t0, 0, NVREGS)
    cp_o0 = pltpu.async_copy(row0_v, o_hbm.at[r0], sem_o0)
    with jax.named_scope("wait_i1"):
        cp_i1.wait()
    with jax.named_scope("max1"):
        acc1 = _max_pass(row1_v, 0, NVREGS,
                         jnp.full((LANES,), -3.4e38, jnp.float32))
    with jax.named_scope("tau1"):
        t1 = _row_tau(row1_v, cand_v, acc1)
    cps1 = []
    for k in range(NCHUNKS):
        _out_pass(row1_v, t1, k * CHUNK_VREGS, (k + 1) * CHUNK_VREGS)
        cps1.append(
            pltpu.async_copy(
                row1_v.at[pl.ds(k * CHUNK_COLS, CHUNK_COLS)],
                o_hbm.at[r1, pl.ds(k * CHUNK_COLS, CHUNK_COLS)],
                sems_c[k],
            )
        )
    cp_o0.wait()
    for cp in cps1:
        cp.wait()


def kernel(inputs):
    return _sparsemax_sc(inputs)


# speedup vs baseline: 1.1737x; 1.0413x over previous
import functools

import jax
import jax.numpy as jnp
from jax import lax
from jax.experimental import pallas as pl
from jax.experimental.pallas import tpu as pltpu
from jax.experimental.pallas import tpu_sc as plsc

ROWS = 64
COLS = 32768
LANES = 16
NUM_CORES = 2
NUM_SUBCORES = 16
NUM_WORKERS = NUM_CORES * NUM_SUBCORES
ROWS_PER_WORKER = ROWS // NUM_WORKERS
NVREGS = COLS // LANES
GROUP_VREGS = 4
GROUP_COLS = GROUP_VREGS * LANES
NGROUPS = COLS // GROUP_COLS
BISECT_ITERS = 15
RESIDENT_VREGS = 16
RESIDENT_COLS = RESIDENT_VREGS * LANES

_MESH = plsc.VectorSubcoreMesh(core_axis_name="c", subcore_axis_name="s")

_GATHER_DNUMS = lax.GatherDimensionNumbers(
    offset_dims=(), collapsed_slice_dims=(0,), start_index_map=(0,)
)


def _shuffle(v, sh):
    idx = jnp.bitwise_xor(lax.iota(jnp.int32, LANES), sh)
    return lax.gather(
        v,
        idx[:, None],
        dimension_numbers=_GATHER_DNUMS,
        slice_sizes=(1,),
        mode=lax.GatherScatterMode.PROMISE_IN_BOUNDS,
    )


def _allmax(v):
    for sh in (8, 4, 2, 1):
        v = jnp.maximum(v, _shuffle(v, sh))
    return v


def _allsum(v):
    for sh in (8, 4, 2, 1):
        v = v + _shuffle(v, sh)
    return v


def _allmax_i(v):
    for sh in (8, 4, 2, 1):
        v = jnp.maximum(v, _shuffle(v, sh))
    return v


def _allmin_i(v):
    for sh in (8, 4, 2, 1):
        v = jnp.minimum(v, _shuffle(v, sh))
    return v


def _max_pass(row_v, lo_vreg, hi_vreg, acc_init):
    @plsc.parallel_loop(lo_vreg, hi_vreg, step=8, unroll=2, carry=acc_init)
    def acc(j, acc_c):
        vs = [row_v[pl.ds((j + k) * LANES, LANES)] for k in range(8)]
        m01 = jnp.maximum(vs[0], vs[1])
        m23 = jnp.maximum(vs[2], vs[3])
        m45 = jnp.maximum(vs[4], vs[5])
        m67 = jnp.maximum(vs[6], vs[7])
        m = jnp.maximum(jnp.maximum(m01, m23), jnp.maximum(m45, m67))
        return jnp.maximum(acc_c, m)

    return acc


def _row_tau(row_v, cand_v, acc):
    f32 = jnp.float32
    maxv = _allmax(acc)
    thr = maxv - 1.0

    lane_id = lax.iota(jnp.int32, LANES)
    sixteen = jnp.full((LANES,), LANES, jnp.int32)
    zero_i = jnp.zeros((LANES,), jnp.int32)

    with jax.named_scope("compact"):
        @plsc.parallel_loop(0, NVREGS, step=1, unroll=8, carry=lane_id)
        def cur16(j, cur_c):
            v = row_v[pl.ds(j * LANES, LANES)]
            m = v > thr
            plsc.store_scatter(cand_v, [cur_c], v, mask=m)
            return cur_c + jnp.where(m, sixteen, zero_i)

    cnt16 = cur16 - lane_id
    targ16 = jnp.maximum(_allmax_i(cnt16), jnp.full((LANES,), RESIDENT_COLS,
                                                    jnp.int32))
    minc16 = _allmin_i(cnt16)
    n_pad = ((targ16 - minc16) // LANES)[0]

    def pad_body(_, cur_c):
        m = cur_c - lane_id < targ16
        plsc.store_scatter(cand_v, [cur_c], thr, mask=m)
        return cur_c + jnp.where(m, sixteen, zero_i)

    lax.fori_loop(0, n_pad, pad_body, cur16)
    nv = (targ16 // LANES)[0]

    cvs = [cand_v[pl.ds(i * LANES, LANES)] for i in range(RESIDENT_VREGS)]

    def bis_body(_, carry):
        lo, hi = carry
        mid = 0.5 * (lo + hi)
        sacc = jnp.zeros((LANES,), f32)
        for cv in cvs:
            sacc = sacc + jnp.maximum(cv - mid, 0.0)

        def s_body(j, sa):
            v = cand_v[pl.ds(j * LANES, LANES)]
            return sa + jnp.maximum(v - mid, 0.0)

        sacc = lax.fori_loop(RESIDENT_VREGS, nv, s_body, sacc)
        pred = _allsum(sacc) >= 1.0
        return jnp.where(pred, mid, lo), jnp.where(pred, hi, mid)

    with jax.named_scope("bisect"):
        lo, _ = lax.fori_loop(0, BISECT_ITERS, bis_body, (thr, maxv))

    sacc = jnp.zeros((LANES,), f32)
    cacc = jnp.zeros((LANES,), f32)
    for cv in cvs:
        m = cv > lo
        sacc = sacc + jnp.where(m, cv, 0.0)
        cacc = cacc + m.astype(f32)

    def rf_body(j, carry):
        sa, ca = carry
        v = cand_v[pl.ds(j * LANES, LANES)]
        m = v > lo
        return sa + jnp.where(m, v, 0.0), ca + m.astype(f32)

    sacc, cacc = lax.fori_loop(RESIDENT_VREGS, nv, rf_body, (sacc, cacc))
    t_vec = (_allsum(sacc) - 1.0) / _allsum(cacc)
    return t_vec


def _out_pass(row_v, t_vec, lo_vreg, hi_vreg):
    @plsc.parallel_loop(lo_vreg, hi_vreg, step=8, unroll=2)
    def _(j):
        for k in range(8):
            v = row_v[pl.ds((j + k) * LANES, LANES)]
            row_v[pl.ds((j + k) * LANES, LANES)] = jnp.maximum(v - t_vec, 0.0)


NCHUNKS = 8
CHUNK_COLS = COLS // NCHUNKS
CHUNK_VREGS = NVREGS // NCHUNKS


@functools.partial(
    pl.kernel,
    out_type=jax.ShapeDtypeStruct((ROWS, COLS), jnp.float32),
    mesh=_MESH,
    scratch_types=[
        pltpu.VMEM((COLS,), jnp.float32),
        pltpu.VMEM((COLS,), jnp.float32),
        pltpu.VMEM((COLS + LANES,), jnp.float32),
        [pltpu.SemaphoreType.DMA] * NCHUNKS,
        pltpu.SemaphoreType.DMA,
        pltpu.SemaphoreType.DMA,
    ],
    compiler_params=pltpu.CompilerParams(needs_layout_passes=False),
)
def _sparsemax_sc(x_hbm, o_hbm, row0_v, row1_v, cand_v,
                  sems_c, sem_i1, sem_o0):
    wid = lax.axis_index("s") * NUM_CORES + lax.axis_index("c")
    r0 = wid * ROWS_PER_WORKER
    r1 = r0 + 1
    cps0 = [
        pltpu.async_copy(
            x_hbm.at[r0, pl.ds(k * CHUNK_COLS, CHUNK_COLS)],
            row0_v.at[pl.ds(k * CHUNK_COLS, CHUNK_COLS)],
            sems_c[k],
        )
        for k in range(NCHUNKS)
    ]
    acc = jnp.full((LANES,), -3.4e38, jnp.float32)
    with jax.named_scope("max0"):
        for k in range(NCHUNKS):
            cps0[k].wait()
            acc = _max_pass(row0_v, k * CHUNK_VREGS, (k + 1) * CHUNK_VREGS, acc)
    cp_i1 = pltpu.async_copy(x_hbm.at[r1], row1_v, sem_i1)
    with jax.named_scope("tau0"):
        t0 = _row_tau(row0_v, cand_v, acc)
    with jax.named_scope("out0"):
        _out_pass(row0_v, t0, 0, NVREGS)
    cp_o0 = pltpu.async_copy(row0_v, o_hbm.at[r0], sem_o0)
    with jax.named_scope("wait_i1"):
        cp_i1.wait()
    with jax.named_scope("max1"):
        acc1 = _max_pass(row1_v, 0, NVREGS,
                         jnp.full((LANES,), -3.4e38, jnp.float32))
    with jax.named_scope("tau1"):
        t1 = _row_tau(row1_v, cand_v, acc1)
    cps1 = []
    for k in range(NCHUNKS):
        _out_pass(row1_v, t1, k * CHUNK_VREGS, (k + 1) * CHUNK_VREGS)
        cps1.append(
            pltpu.async_copy(
                row1_v.at[pl.ds(k * CHUNK_COLS, CHUNK_COLS)],
                o_hbm.at[r1, pl.ds(k * CHUNK_COLS, CHUNK_COLS)],
                sems_c[k],
            )
        )
    cp_o0.wait()
    for cp in cps1:
        cp.wait()


def kernel(inputs):
    return _sparsemax_sc(inputs)
